# Initial kernel scaffold; baseline (speedup 1.0000x reference)
import jax
import jax.numpy as jnp
from jax.experimental import pallas as pl


def kernel(atom_feature, edge_feature, edge_index, W_full, b_full, g_msg, b_msg, g_agg, b_agg):
    raise NotImplementedError("write your pallas kernel here")


# speedup vs baseline: 2.5752x; 2.5752x over previous
import functools

import jax
import jax.numpy as jnp
from jax import lax
from jax.experimental import pallas as pl
from jax.experimental.pallas import tpu as pltpu
from jax.experimental.pallas import tpu_sc as plsc

F = 128
BW = 16
H = 2 * F
EPS = 1e-5
NC = 2
NS = 16
NW = NC * NS
CH = 80
TE = 3200

_sc_mesh = plsc.VectorSubcoreMesh(core_axis_name="c", subcore_axis_name="s")


def _precompute_body(atom_ref, w1_ref, w2_ref, a1_ref, a2_ref):
    a = atom_ref[...]
    a1_ref[...] = jnp.dot(a, w1_ref[...], preferred_element_type=jnp.float32)
    a2_ref[...] = jnp.dot(a, w2_ref[...], preferred_element_type=jnp.float32)


@functools.lru_cache(maxsize=None)
def _make_gather_h(n, e):
    epw = e // NW
    nch = epw // CH

    @functools.partial(
        pl.kernel,
        out_type=jax.ShapeDtypeStruct((e, H), jnp.float32),
        mesh=_sc_mesh,
        scratch_types=[
            pltpu.VMEM((nch, CH), jnp.int32),
            pltpu.VMEM((nch, CH), jnp.int32),
            pltpu.VMEM((CH, H), jnp.float32),
            pltpu.VMEM((CH, H), jnp.float32),
            pltpu.SemaphoreType.DMA,
            pltpu.SemaphoreType.DMA,
        ],
    )
    def gather_h(a1_hbm, a2_hbm, src_hbm, dst_hbm, h_hbm, src_v, dst_v, a1b, a2b, sm1, sm2):
        cc = lax.axis_index("c")
        ss = lax.axis_index("s")
        wid = ss * NC + cc
        pltpu.sync_copy(src_hbm.at[wid], src_v)
        pltpu.sync_copy(dst_hbm.at[wid], dst_v)
        base = wid * epw

        @pl.loop(0, nch)
        def _chunk(ch):
            d1 = pltpu.async_copy(a1_hbm.at[src_v.at[ch]], a1b, sm1)
            d2 = pltpu.async_copy(a2_hbm.at[dst_v.at[ch]], a2b, sm2)
            d1.wait()
            d2.wait()

            @pl.loop(0, CH)
            def _row(i):
                for j in range(H // 16):
                    sl = pl.ds(j * 16, 16)
                    a1b[i, sl] += a2b[i, sl]

            pltpu.sync_copy(a1b, h_hbm.at[pl.ds(base + ch * CH, CH)])

    return gather_h


def _stats_body(h_ref, x2_ref, w3k_ref, b_ref, st_ref):
    g = jnp.dot(x2_ref[...], w3k_ref[...], preferred_element_type=jnp.float32)
    g = h_ref[...] + g.reshape(TE, H) + b_ref[...]

    @pl.when(pl.program_id(0) == 0)
    def _():
        st_ref[...] = jnp.zeros_like(st_ref)

    su = jnp.sum(g, axis=0, keepdims=True)
    sq = jnp.sum(g * g, axis=0, keepdims=True)
    st_ref[...] += jnp.concatenate([su, sq, jnp.zeros((6, H), jnp.float32)], axis=0)


def _act_body(h_ref, x2_ref, w3k_ref, b_ref, ss_ref, m_ref):
    g = jnp.dot(x2_ref[...], w3k_ref[...], preferred_element_type=jnp.float32)
    g = h_ref[...] + g.reshape(TE, H) + b_ref[...]
    g = g * ss_ref[0:1, :] + ss_ref[1:2, :]
    m_ref[...] = jax.nn.sigmoid(g[:, :F]) * jax.nn.softplus(g[:, F:])


@functools.lru_cache(maxsize=None)
def _make_scatter(n, e):
    epw = e // NW
    nch = epw // CH
    npt0 = (n // NS) & ~7
    nlast = n - npt0 * (NS - 1)

    @functools.partial(
        pl.kernel,
        out_type=jax.ShapeDtypeStruct((NC, n, F), jnp.float32),
        mesh=_sc_mesh,
        scratch_types=[
            pltpu.VMEM((nch, CH), jnp.int32),
            pltpu.VMEM((CH, F), jnp.float32),
            pltpu.VMEM_SHARED((n, F), jnp.float32),
            pltpu.SemaphoreType.DMA,
        ],
    )
    def scatter_m(m_hbm, src_hbm, z_hbm, out_hbm, src_v, mb, acc, sm):
        cc = lax.axis_index("c")
        ss = lax.axis_index("s")
        wid = ss * NC + cc
        pltpu.sync_copy(src_hbm.at[wid], src_v)
        row0 = ss * npt0

        @pl.when(ss < NS - 1)
        def _():
            pltpu.sync_copy(z_hbm.at[pl.ds(row0, npt0)], acc.at[pl.ds(row0, npt0)])

        @pl.when(ss == NS - 1)
        def _():
            pltpu.sync_copy(z_hbm.at[pl.ds(row0, nlast)], acc.at[pl.ds(row0, nlast)])

        plsc.subcore_barrier()
        base = wid * epw

        @pl.loop(0, nch)
        def _chunk(ch):
            pltpu.sync_copy(m_hbm.at[pl.ds(base + ch * CH, CH)], mb)
            pltpu.sync_copy(mb, acc.at[src_v.at[ch]], add=True)

        plsc.subcore_barrier()

        @pl.when(ss < NS - 1)
        def _():
            pltpu.sync_copy(acc.at[pl.ds(row0, npt0)], out_hbm.at[cc, pl.ds(row0, npt0)])

        @pl.when(ss == NS - 1)
        def _():
            pltpu.sync_copy(acc.at[pl.ds(row0, nlast)], out_hbm.at[cc, pl.ds(row0, nlast)])

    return scatter_m


def _final_body(p_ref, atom_ref, ga_ref, ba_ref, out_ref):
    mi = p_ref[0] + p_ref[1]
    mean = jnp.mean(mi, axis=0, keepdims=True)
    ctr = mi - mean
    var = jnp.mean(ctr * ctr, axis=0, keepdims=True)
    mih = ctr * lax.rsqrt(var + EPS) * ga_ref[...] + ba_ref[...]
    out_ref[...] = jax.nn.softplus(atom_ref[...] + mih)


def kernel(atom_feature, edge_feature, edge_index, W_full, b_full, g_msg, b_msg, g_agg, b_agg):
    n, f = atom_feature.shape
    e = edge_feature.shape[0]
    assert f == F and edge_feature.shape[1] == BW
    assert e % (NW * CH) == 0 and n % NS == 0
    epw = e // NW
    nch = epw // CH

    src = edge_index[:, 0].astype(jnp.int32)
    dst = edge_index[:, 1].astype(jnp.int32)
    src3 = src.reshape(NW, nch, CH)
    dst3 = dst.reshape(NW, nch, CH)
    W1 = W_full[:F]
    W2 = W_full[F : 2 * F]
    W3 = W_full[2 * F :]
    w3k = jnp.kron(jnp.eye(8, dtype=jnp.float32), W3)
    x2 = edge_feature.reshape(e // 8, 8 * BW)
    b2 = b_full.reshape(1, H)

    A1, A2 = pl.pallas_call(
        _precompute_body,
        out_shape=[jax.ShapeDtypeStruct((n, H), jnp.float32)] * 2,
    )(atom_feature, W1, W2)

    h = _make_gather_h(n, e)(A1, A2, src3, dst3)

    grid = (e // TE,)
    stats = pl.pallas_call(
        _stats_body,
        grid=grid,
        in_specs=[
            pl.BlockSpec((TE, H), lambda i: (i, 0)),
            pl.BlockSpec((TE // 8, 8 * BW), lambda i: (i, 0)),
            pl.BlockSpec((F, 8 * H), lambda i: (0, 0)),
            pl.BlockSpec((1, H), lambda i: (0, 0)),
        ],
        out_specs=pl.BlockSpec((8, H), lambda i: (0, 0)),
        out_shape=jax.ShapeDtypeStruct((8, H), jnp.float32),
    )(h, x2, w3k, b2)

    mean = stats[0] / e
    var = stats[1] / e - mean * mean
    scale = g_msg / jnp.sqrt(var + EPS)
    shift = b_msg - mean * scale
    ssb = jnp.zeros((8, H), jnp.float32).at[0].set(scale).at[1].set(shift)

    m = pl.pallas_call(
        _act_body,
        grid=grid,
        in_specs=[
            pl.BlockSpec((TE, H), lambda i: (i, 0)),
            pl.BlockSpec((TE // 8, 8 * BW), lambda i: (i, 0)),
            pl.BlockSpec((F, 8 * H), lambda i: (0, 0)),
            pl.BlockSpec((1, H), lambda i: (0, 0)),
            pl.BlockSpec((8, H), lambda i: (0, 0)),
        ],
        out_specs=pl.BlockSpec((TE, F), lambda i: (i, 0)),
        out_shape=jax.ShapeDtypeStruct((e, F), jnp.float32),
    )(h, x2, w3k, b2, ssb)

    zeros_nf = jnp.zeros((n, F), jnp.float32)
    partials = _make_scatter(n, e)(m, src3, zeros_nf)

    out = pl.pallas_call(
        _final_body,
        out_shape=jax.ShapeDtypeStruct((n, F), jnp.float32),
    )(partials, atom_feature, g_agg.reshape(1, F), b_agg.reshape(1, F))
    return out


# speedup vs baseline: 3.2048x; 1.2445x over previous
import functools

import jax
import jax.numpy as jnp
from jax import lax
from jax.experimental import pallas as pl
from jax.experimental.pallas import tpu as pltpu
from jax.experimental.pallas import tpu_sc as plsc

F = 128
BW = 16
H = 2 * F
EPS = 1e-5
NC = 2
NS = 16
NW = NC * NS
CH = 80
TE = 3200

_sc_mesh = plsc.VectorSubcoreMesh(core_axis_name="c", subcore_axis_name="s")


def _precompute_body(atom_ref, w1_ref, w2_ref, a1_ref, a2_ref):
    a = atom_ref[...]

    def pack(w_ref):
        v = jnp.dot(a, w_ref[...], preferred_element_type=jnp.float32)
        lo = lax.bitcast_convert_type(v[:, :F].astype(jnp.bfloat16), jnp.uint16)
        hi = lax.bitcast_convert_type(v[:, F:].astype(jnp.bfloat16), jnp.uint16)
        word = lo.astype(jnp.uint32) | (hi.astype(jnp.uint32) << 16)
        return lax.bitcast_convert_type(word, jnp.int32)

    a1_ref[...] = pack(w1_ref)
    a2_ref[...] = pack(w2_ref)


NSLOT = 4


@functools.lru_cache(maxsize=None)
def _make_gather_h(n, e):
    epw = e // NW
    nch = epw // CH
    assert nch >= NSLOT

    @functools.partial(
        pl.kernel,
        out_type=[jax.ShapeDtypeStruct((e, F), jnp.int32)] * 2,
        mesh=_sc_mesh,
        scratch_types=[
            pltpu.VMEM((nch, CH), jnp.int32),
            pltpu.VMEM((nch, CH), jnp.int32),
        ]
        + [pltpu.VMEM((CH, F), jnp.int32)] * (2 * NSLOT)
        + [pltpu.SemaphoreType.DMA] * (2 * NSLOT),
    )
    def gather_h(a1_hbm, a2_hbm, src_hbm, dst_hbm, h1_hbm, h2_hbm, src_v, dst_v, *scr):
        bufs1 = scr[0:NSLOT]
        bufs2 = scr[NSLOT : 2 * NSLOT]
        sg = scr[2 * NSLOT : 3 * NSLOT]
        sw = scr[3 * NSLOT : 4 * NSLOT]
        cc = lax.axis_index("c")
        ss = lax.axis_index("s")
        wid = ss * NC + cc
        pltpu.sync_copy(src_hbm.at[wid], src_v)
        pltpu.sync_copy(dst_hbm.at[wid], dst_v)
        base = wid * epw

        def issue_g(ch, s):
            pltpu.async_copy(a1_hbm.at[src_v.at[ch]], bufs1[s], sg[s])
            pltpu.async_copy(a2_hbm.at[dst_v.at[ch]], bufs2[s], sg[s])

        def wait_g(s):
            pltpu.make_async_copy(a1_hbm.at[src_v.at[0]], bufs1[s], sg[s]).wait()
            pltpu.make_async_copy(a2_hbm.at[dst_v.at[0]], bufs2[s], sg[s]).wait()

        def issue_w(ch, s):
            rows = pl.ds(base + ch * CH, CH)
            pltpu.async_copy(bufs1[s], h1_hbm.at[rows], sw[s])
            pltpu.async_copy(bufs2[s], h2_hbm.at[rows], sw[s])

        def wait_w(s):
            rows = pl.ds(base, CH)
            pltpu.make_async_copy(bufs1[s], h1_hbm.at[rows], sw[s]).wait()
            pltpu.make_async_copy(bufs2[s], h2_hbm.at[rows], sw[s]).wait()

        for s in range(NSLOT - 1):
            issue_g(s, s)

        @pl.loop(0, pl.cdiv(nch, NSLOT))
        def _quad(p):
            c0 = p * NSLOT
            for j in range(NSLOT):
                s = j
                c = c0 + j

                def _step(c=c, s=s, first=(j == 0)):
                    wait_g(s)
                    issue_w(c, s)
                    cn = c + (NSLOT - 1)
                    sn = (s + NSLOT - 1) % NSLOT

                    @pl.when(cn < nch)
                    def _():
                        if first:
                            pl.when(c >= 1)(lambda: wait_w(sn))
                        else:
                            wait_w(sn)
                        issue_g(cn, sn)

                if j == 0:
                    _step()
                else:
                    pl.when(c < nch)(_step)

        for s in range(NSLOT):
            wait_w(s)

    return gather_h


def _unpack_halves(hh):
    hf = lax.bitcast_convert_type(hh << 16, jnp.float32)
    hs = lax.bitcast_convert_type(hh & jnp.int32(-65536), jnp.float32)
    return hf, hs


def _stats_body(h1_ref, h2_ref, x2_ref, w3k_ref, b_ref, st_ref):
    h1f, h1s = _unpack_halves(h1_ref[...])
    h2f, h2s = _unpack_halves(h2_ref[...])
    ge = jnp.dot(x2_ref[...], w3k_ref[...], preferred_element_type=jnp.float32)
    ge = ge.reshape(TE, H)
    gf = h1f + h2f + ge[:, :F] + b_ref[:, :F]
    gs = h1s + h2s + ge[:, F:] + b_ref[:, F:]

    @pl.when(pl.program_id(0) == 0)
    def _():
        st_ref[...] = jnp.zeros_like(st_ref)

    su = jnp.concatenate(
        [jnp.sum(gf, axis=0, keepdims=True), jnp.sum(gs, axis=0, keepdims=True)], axis=1
    )
    sq = jnp.concatenate(
        [jnp.sum(gf * gf, axis=0, keepdims=True), jnp.sum(gs * gs, axis=0, keepdims=True)],
        axis=1,
    )
    st_ref[...] += jnp.concatenate([su, sq, jnp.zeros((6, H), jnp.float32)], axis=0)


def _act_body(h1_ref, h2_ref, x2_ref, w3k_ref, b_ref, ss_ref, m_ref):
    h1f, h1s = _unpack_halves(h1_ref[...])
    h2f, h2s = _unpack_halves(h2_ref[...])
    ge = jnp.dot(x2_ref[...], w3k_ref[...], preferred_element_type=jnp.float32)
    ge = ge.reshape(TE, H)
    gf = h1f + h2f + ge[:, :F] + b_ref[:, :F]
    gs = h1s + h2s + ge[:, F:] + b_ref[:, F:]
    gf = gf * ss_ref[0:1, :F] + ss_ref[1:2, :F]
    gs = gs * ss_ref[0:1, F:] + ss_ref[1:2, F:]
    m_ref[...] = jax.nn.sigmoid(gf) * jax.nn.softplus(gs)


@functools.lru_cache(maxsize=None)
def _make_scatter(n, e):
    epw = e // NW
    nch = epw // CH
    npt0 = (n // NS) & ~7
    nlast = n - npt0 * (NS - 1)

    @functools.partial(
        pl.kernel,
        out_type=jax.ShapeDtypeStruct((NC, n, F), jnp.float32),
        mesh=_sc_mesh,
        scratch_types=[
            pltpu.VMEM((nch, CH), jnp.int32),
            pltpu.VMEM((CH, F), jnp.float32),
            pltpu.VMEM_SHARED((n, F), jnp.float32),
            pltpu.SemaphoreType.DMA,
        ],
    )
    def scatter_m(m_hbm, src_hbm, z_hbm, out_hbm, src_v, mb, acc, sm):
        cc = lax.axis_index("c")
        ss = lax.axis_index("s")
        wid = ss * NC + cc
        pltpu.sync_copy(src_hbm.at[wid], src_v)
        row0 = ss * npt0

        @pl.when(ss < NS - 1)
        def _():
            pltpu.sync_copy(z_hbm.at[pl.ds(row0, npt0)], acc.at[pl.ds(row0, npt0)])

        @pl.when(ss == NS - 1)
        def _():
            pltpu.sync_copy(z_hbm.at[pl.ds(row0, nlast)], acc.at[pl.ds(row0, nlast)])

        plsc.subcore_barrier()
        base = wid * epw

        @pl.loop(0, nch)
        def _chunk(ch):
            pltpu.sync_copy(m_hbm.at[pl.ds(base + ch * CH, CH)], mb)
            pltpu.sync_copy(mb, acc.at[src_v.at[ch]], add=True)

        plsc.subcore_barrier()

        @pl.when(ss < NS - 1)
        def _():
            pltpu.sync_copy(acc.at[pl.ds(row0, npt0)], out_hbm.at[cc, pl.ds(row0, npt0)])

        @pl.when(ss == NS - 1)
        def _():
            pltpu.sync_copy(acc.at[pl.ds(row0, nlast)], out_hbm.at[cc, pl.ds(row0, nlast)])

    return scatter_m


def _final_body(p_ref, atom_ref, ga_ref, ba_ref, out_ref):
    mi = p_ref[0] + p_ref[1]
    mean = jnp.mean(mi, axis=0, keepdims=True)
    ctr = mi - mean
    var = jnp.mean(ctr * ctr, axis=0, keepdims=True)
    mih = ctr * lax.rsqrt(var + EPS) * ga_ref[...] + ba_ref[...]
    out_ref[...] = jax.nn.softplus(atom_ref[...] + mih)


def kernel(atom_feature, edge_feature, edge_index, W_full, b_full, g_msg, b_msg, g_agg, b_agg):
    n, f = atom_feature.shape
    e = edge_feature.shape[0]
    assert f == F and edge_feature.shape[1] == BW
    assert e % (NW * CH) == 0 and e % TE == 0
    epw = e // NW
    nch = epw // CH

    src = edge_index[:, 0].astype(jnp.int32)
    dst = edge_index[:, 1].astype(jnp.int32)
    src3 = src.reshape(NW, nch, CH)
    dst3 = dst.reshape(NW, nch, CH)
    W1 = W_full[:F]
    W2 = W_full[F : 2 * F]
    W3 = W_full[2 * F :]
    w3k = jnp.kron(jnp.eye(8, dtype=jnp.float32), W3)
    x2 = edge_feature.reshape(e // 8, 8 * BW)
    b2 = b_full.reshape(1, H)

    A1, A2 = pl.pallas_call(
        _precompute_body,
        out_shape=[jax.ShapeDtypeStruct((n, F), jnp.int32)] * 2,
    )(atom_feature, W1, W2)

    h1, h2 = _make_gather_h(n, e)(A1, A2, src3, dst3)

    grid = (e // TE,)
    stats = pl.pallas_call(
        _stats_body,
        grid=grid,
        in_specs=[
            pl.BlockSpec((TE, F), lambda i: (i, 0)),
            pl.BlockSpec((TE, F), lambda i: (i, 0)),
            pl.BlockSpec((TE // 8, 8 * BW), lambda i: (i, 0)),
            pl.BlockSpec((F, 8 * H), lambda i: (0, 0)),
            pl.BlockSpec((1, H), lambda i: (0, 0)),
        ],
        out_specs=pl.BlockSpec((8, H), lambda i: (0, 0)),
        out_shape=jax.ShapeDtypeStruct((8, H), jnp.float32),
    )(h1, h2, x2, w3k, b2)

    mean = stats[0] / e
    var = stats[1] / e - mean * mean
    scale = g_msg / jnp.sqrt(var + EPS)
    shift = b_msg - mean * scale
    ssb = jnp.zeros((8, H), jnp.float32).at[0].set(scale).at[1].set(shift)

    m = pl.pallas_call(
        _act_body,
        grid=grid,
        in_specs=[
            pl.BlockSpec((TE, F), lambda i: (i, 0)),
            pl.BlockSpec((TE, F), lambda i: (i, 0)),
            pl.BlockSpec((TE // 8, 8 * BW), lambda i: (i, 0)),
            pl.BlockSpec((F, 8 * H), lambda i: (0, 0)),
            pl.BlockSpec((1, H), lambda i: (0, 0)),
            pl.BlockSpec((8, H), lambda i: (0, 0)),
        ],
        out_specs=pl.BlockSpec((TE, F), lambda i: (i, 0)),
        out_shape=jax.ShapeDtypeStruct((e, F), jnp.float32),
    )(h1, h2, x2, w3k, b2, ssb)

    zeros_nf = jnp.zeros((n, F), jnp.float32)
    partials = _make_scatter(n, e)(m, src3, zeros_nf)

    out = pl.pallas_call(
        _final_body,
        out_shape=jax.ShapeDtypeStruct((n, F), jnp.float32),
    )(partials, atom_feature, g_agg.reshape(1, F), b_agg.reshape(1, F))
    return out


# speedup vs baseline: 3.6126x; 1.1272x over previous
import functools

import jax
import jax.numpy as jnp
from jax import lax
from jax.experimental import pallas as pl
from jax.experimental.pallas import tpu as pltpu
from jax.experimental.pallas import tpu_sc as plsc

F = 128
BW = 16
H = 2 * F
EPS = 1e-5
NC = 2
NS = 16
NW = NC * NS
TE = 3200
NSPLIT = 2
NSLOT = 4

_sc_mesh = plsc.VectorSubcoreMesh(core_axis_name="c", subcore_axis_name="s")


def _precompute_body(atom_ref, w1_ref, w2_ref, a1_ref, a2_ref):
    a = atom_ref[...]

    def pack(w_ref):
        v = jnp.dot(a, w_ref[...], preferred_element_type=jnp.float32)
        lo = lax.bitcast_convert_type(v[:, :F].astype(jnp.bfloat16), jnp.uint16)
        hi = lax.bitcast_convert_type(v[:, F:].astype(jnp.bfloat16), jnp.uint16)
        word = lo.astype(jnp.uint32) | (hi.astype(jnp.uint32) << 16)
        return lax.bitcast_convert_type(word, jnp.int32)

    a1_ref[...] = pack(w1_ref)
    a2_ref[...] = pack(w2_ref)


@functools.lru_cache(maxsize=None)
def _make_gather_h(eh, ch):
    epw = eh // NW
    nch = epw // ch
    assert epw % ch == 0 and nch >= NSLOT

    @functools.partial(
        pl.kernel,
        out_type=[jax.ShapeDtypeStruct((eh, F), jnp.int32)] * 2,
        mesh=_sc_mesh,
        scratch_types=[
            pltpu.VMEM((nch, ch), jnp.int32),
            pltpu.VMEM((nch, ch), jnp.int32),
        ]
        + [pltpu.VMEM((ch, F), jnp.int32)] * (2 * NSLOT)
        + [pltpu.SemaphoreType.DMA] * (2 * NSLOT),
    )
    def gather_h(a1_hbm, a2_hbm, src_hbm, dst_hbm, h1_hbm, h2_hbm, src_v, dst_v, *scr):
        bufs1 = scr[0:NSLOT]
        bufs2 = scr[NSLOT : 2 * NSLOT]
        sg = scr[2 * NSLOT : 3 * NSLOT]
        sw = scr[3 * NSLOT : 4 * NSLOT]
        cc = lax.axis_index("c")
        ss = lax.axis_index("s")
        wid = ss * NC + cc
        pltpu.sync_copy(src_hbm.at[wid], src_v)
        pltpu.sync_copy(dst_hbm.at[wid], dst_v)
        base = wid * epw

        def issue_g(c, s):
            pltpu.async_copy(a1_hbm.at[src_v.at[c]], bufs1[s], sg[s])
            pltpu.async_copy(a2_hbm.at[dst_v.at[c]], bufs2[s], sg[s])

        def wait_g(s):
            pltpu.make_async_copy(a1_hbm.at[src_v.at[0]], bufs1[s], sg[s]).wait()
            pltpu.make_async_copy(a2_hbm.at[dst_v.at[0]], bufs2[s], sg[s]).wait()

        def issue_w(c, s):
            rows = pl.ds(base + c * ch, ch)
            pltpu.async_copy(bufs1[s], h1_hbm.at[rows], sw[s])
            pltpu.async_copy(bufs2[s], h2_hbm.at[rows], sw[s])

        def wait_w(s):
            rows = pl.ds(base, ch)
            pltpu.make_async_copy(bufs1[s], h1_hbm.at[rows], sw[s]).wait()
            pltpu.make_async_copy(bufs2[s], h2_hbm.at[rows], sw[s]).wait()

        for s in range(NSLOT - 1):
            issue_g(s, s)

        @pl.loop(0, pl.cdiv(nch, NSLOT))
        def _quad(p):
            c0 = p * NSLOT
            for j in range(NSLOT):
                s = j
                c = c0 + j

                def _step(c=c, s=s, first=(j == 0)):
                    wait_g(s)
                    issue_w(c, s)
                    cn = c + (NSLOT - 1)
                    sn = (s + NSLOT - 1) % NSLOT

                    @pl.when(cn < nch)
                    def _():
                        if first:
                            pl.when(c >= 1)(lambda: wait_w(sn))
                        else:
                            wait_w(sn)
                        issue_g(cn, sn)

                if j == 0:
                    _step()
                else:
                    pl.when(c < nch)(_step)

        for s in range(NSLOT):
            wait_w(s)

    return gather_h


def _unpack_halves(hh):
    hf = lax.bitcast_convert_type(hh << 16, jnp.float32)
    hs = lax.bitcast_convert_type(hh & jnp.int32(-65536), jnp.float32)
    return hf, hs


def _stats_body(h1_ref, h2_ref, x2_ref, w3k_ref, b_ref, st_ref):
    h1f, h1s = _unpack_halves(h1_ref[...])
    h2f, h2s = _unpack_halves(h2_ref[...])
    ge = jnp.dot(x2_ref[...], w3k_ref[...], preferred_element_type=jnp.float32)
    ge = ge.reshape(TE, H)
    gf = h1f + h2f + ge[:, :F] + b_ref[:, :F]
    gs = h1s + h2s + ge[:, F:] + b_ref[:, F:]

    @pl.when(pl.program_id(0) == 0)
    def _():
        st_ref[...] = jnp.zeros_like(st_ref)

    su = jnp.concatenate(
        [jnp.sum(gf, axis=0, keepdims=True), jnp.sum(gs, axis=0, keepdims=True)], axis=1
    )
    sq = jnp.concatenate(
        [jnp.sum(gf * gf, axis=0, keepdims=True), jnp.sum(gs * gs, axis=0, keepdims=True)],
        axis=1,
    )
    st_ref[...] += jnp.concatenate([su, sq, jnp.zeros((6, H), jnp.float32)], axis=0)


def _act_body(h1_ref, h2_ref, x2_ref, w3k_ref, b_ref, ss_ref, m_ref):
    h1f, h1s = _unpack_halves(h1_ref[...])
    h2f, h2s = _unpack_halves(h2_ref[...])
    ge = jnp.dot(x2_ref[...], w3k_ref[...], preferred_element_type=jnp.float32)
    ge = ge.reshape(TE, H)
    gf = h1f + h2f + ge[:, :F] + b_ref[:, :F]
    gs = h1s + h2s + ge[:, F:] + b_ref[:, F:]
    gf = gf * ss_ref[0:1, :F] + ss_ref[1:2, :F]
    gs = gs * ss_ref[0:1, F:] + ss_ref[1:2, F:]
    m_ref[...] = jax.nn.sigmoid(gf) * jax.nn.softplus(gs)


@functools.lru_cache(maxsize=None)
def _make_scatter(n, eh, ch):
    epw = eh // NW
    nch = epw // ch
    assert epw % ch == 0
    npt0 = (n // NS) & ~7
    nlast = n - npt0 * (NS - 1)

    @functools.partial(
        pl.kernel,
        out_type=jax.ShapeDtypeStruct((NC, n, F), jnp.float32),
        mesh=_sc_mesh,
        scratch_types=[
            pltpu.VMEM((nch, ch), jnp.int32),
            pltpu.VMEM((ch, F), jnp.float32),
            pltpu.VMEM((ch, F), jnp.float32),
            pltpu.VMEM_SHARED((n, F), jnp.float32),
            pltpu.SemaphoreType.DMA,
            pltpu.SemaphoreType.DMA,
        ],
    )
    def scatter_m(m_hbm, src_hbm, z_hbm, out_hbm, src_v, mb0, mb1, acc, sm0, sm1):
        mbs = (mb0, mb1)
        sms = (sm0, sm1)
        cc = lax.axis_index("c")
        ss = lax.axis_index("s")
        wid = ss * NC + cc
        pltpu.sync_copy(src_hbm.at[wid], src_v)
        row0 = ss * npt0

        @pl.when(ss < NS - 1)
        def _():
            pltpu.sync_copy(z_hbm.at[pl.ds(row0, npt0)], acc.at[pl.ds(row0, npt0)])

        @pl.when(ss == NS - 1)
        def _():
            pltpu.sync_copy(z_hbm.at[pl.ds(row0, nlast)], acc.at[pl.ds(row0, nlast)])

        plsc.subcore_barrier()
        base = wid * epw

        def issue_r(c, s):
            pltpu.async_copy(m_hbm.at[pl.ds(base + c * ch, ch)], mbs[s], sms[s])

        def wait_r(s):
            pltpu.make_async_copy(m_hbm.at[pl.ds(base, ch)], mbs[s], sms[s]).wait()

        issue_r(0, 0)

        @pl.loop(0, pl.cdiv(nch, 2))
        def _pair(p):
            c0 = p * 2
            for j in range(2):
                c = c0 + j

                def _step(c=c, s=j):
                    @pl.when(c + 1 < nch)
                    def _():
                        issue_r(c + 1, 1 - s)

                    wait_r(s)
                    pltpu.sync_copy(mbs[s], acc.at[src_v.at[c]], add=True)

                if j == 0:
                    _step()
                else:
                    pl.when(c < nch)(_step)

        plsc.subcore_barrier()

        @pl.when(ss < NS - 1)
        def _():
            pltpu.sync_copy(acc.at[pl.ds(row0, npt0)], out_hbm.at[cc, pl.ds(row0, npt0)])

        @pl.when(ss == NS - 1)
        def _():
            pltpu.sync_copy(acc.at[pl.ds(row0, nlast)], out_hbm.at[cc, pl.ds(row0, nlast)])

    return scatter_m


def _final_body(pa_ref, pb_ref, atom_ref, ga_ref, ba_ref, out_ref):
    mi = pa_ref[0] + pa_ref[1] + pb_ref[0] + pb_ref[1]
    mean = jnp.mean(mi, axis=0, keepdims=True)
    ctr = mi - mean
    var = jnp.mean(ctr * ctr, axis=0, keepdims=True)
    mih = ctr * lax.rsqrt(var + EPS) * ga_ref[...] + ba_ref[...]
    out_ref[...] = jax.nn.softplus(atom_ref[...] + mih)


def kernel(atom_feature, edge_feature, edge_index, W_full, b_full, g_msg, b_msg, g_agg, b_agg):
    n, f = atom_feature.shape
    e = edge_feature.shape[0]
    eh = e // NSPLIT
    ch = 40
    assert f == F and edge_feature.shape[1] == BW
    assert eh % (NW * ch) == 0 and eh % TE == 0 and n % NS == 0
    epw = eh // NW
    nch = epw // ch

    src = edge_index[:, 0].astype(jnp.int32)
    dst = edge_index[:, 1].astype(jnp.int32)
    W1 = W_full[:F]
    W2 = W_full[F : 2 * F]
    W3 = W_full[2 * F :]
    w3k = jnp.kron(jnp.eye(8, dtype=jnp.float32), W3)
    x2 = edge_feature.reshape(e // 8, 8 * BW)
    b2 = b_full.reshape(1, H)

    A1, A2 = pl.pallas_call(
        _precompute_body,
        out_shape=[jax.ShapeDtypeStruct((n, F), jnp.int32)] * 2,
    )(atom_feature, W1, W2)

    gather = _make_gather_h(eh, ch)
    halves = []
    for half in range(NSPLIT):
        src3 = lax.slice_in_dim(src, half * eh, (half + 1) * eh).reshape(NW, nch, ch)
        dst3 = lax.slice_in_dim(dst, half * eh, (half + 1) * eh).reshape(NW, nch, ch)
        h1, h2 = gather(A1, A2, src3, dst3)
        halves.append((src3, h1, h2))

    grid = (eh // TE,)
    stats = []
    for half, (_, h1, h2) in enumerate(halves):
        xoff = half * (eh // TE)
        st = pl.pallas_call(
            _stats_body,
            grid=grid,
            in_specs=[
                pl.BlockSpec((TE, F), lambda i: (i, 0)),
                pl.BlockSpec((TE, F), lambda i: (i, 0)),
                pl.BlockSpec((TE // 8, 8 * BW), lambda i, xoff=xoff: (i + xoff, 0)),
                pl.BlockSpec((F, 8 * H), lambda i: (0, 0)),
                pl.BlockSpec((1, H), lambda i: (0, 0)),
            ],
            out_specs=pl.BlockSpec((8, H), lambda i: (0, 0)),
            out_shape=jax.ShapeDtypeStruct((8, H), jnp.float32),
        )(h1, h2, x2, w3k, b2)
        stats.append(st)

    st = stats[0] + stats[1]
    mean = st[0] / e
    var = st[1] / e - mean * mean
    scale = g_msg / jnp.sqrt(var + EPS)
    shift = b_msg - mean * scale
    ssb = jnp.zeros((8, H), jnp.float32).at[0].set(scale).at[1].set(shift)

    scatter = _make_scatter(n, eh, ch)
    zeros_nf = jnp.zeros((n, F), jnp.float32)
    partials = []
    for half, (src3, h1, h2) in enumerate(halves):
        xoff = half * (eh // TE)
        m = pl.pallas_call(
            _act_body,
            grid=grid,
            in_specs=[
                pl.BlockSpec((TE, F), lambda i: (i, 0)),
                pl.BlockSpec((TE, F), lambda i: (i, 0)),
                pl.BlockSpec((TE // 8, 8 * BW), lambda i, xoff=xoff: (i + xoff, 0)),
                pl.BlockSpec((F, 8 * H), lambda i: (0, 0)),
                pl.BlockSpec((1, H), lambda i: (0, 0)),
                pl.BlockSpec((8, H), lambda i: (0, 0)),
            ],
            out_specs=pl.BlockSpec((TE, F), lambda i: (i, 0)),
            out_shape=jax.ShapeDtypeStruct((eh, F), jnp.float32),
        )(h1, h2, x2, w3k, b2, ssb)
        partials.append(scatter(m, src3, zeros_nf))

    out = pl.pallas_call(
        _final_body,
        out_shape=jax.ShapeDtypeStruct((n, F), jnp.float32),
    )(partials[0], partials[1], atom_feature, g_agg.reshape(1, F), b_agg.reshape(1, F))
    return out
